# baseline (speedup 1.0000x reference)
import functools

import jax
import jax.numpy as jnp
from jax import lax
from jax.experimental import pallas as pl
from jax.experimental.pallas import tpu as pltpu
from jax.experimental.pallas import tpu_sc as plsc

_D = 64
_B, _L = 4096, 200
_N = _B * _L

_NC = 2
_NS = 16
_NW = _NC * _NS
_PER_W = _N // _NW
_IDXW = 128
_CROWS = 8
_CHUNK = _CROWS * _IDXW
_NCHUNK = _PER_W // _CHUNK

_mesh = plsc.VectorSubcoreMesh(core_axis_name="c", subcore_axis_name="s")


@functools.partial(
    pl.kernel,
    out_type=jax.ShapeDtypeStruct((_N, _D), jnp.float32),
    mesh=_mesh,
    compiler_params=pltpu.CompilerParams(use_tc_tiling_on_sc=False),
    scratch_types=[
        pltpu.VMEM((_CROWS, _IDXW), jnp.int32),
        pltpu.VMEM((_CHUNK, _D), jnp.float32),
        pltpu.SemaphoreType.DMA,
    ],
)
def _emb_lookup(table_hbm, idx_hbm, out_hbm, idx_v, rows_v, sem):
    wid = lax.axis_index("s") * _NC + lax.axis_index("c")
    row0 = wid * (_PER_W // _IDXW)
    out0 = wid * _PER_W

    def chunk_body(i, carry):
        r = row0 + i * _CROWS
        pltpu.sync_copy(idx_hbm.at[pl.ds(r, _CROWS)], idx_v)
        copies = []
        for j in range(_CROWS):
            copies.append(
                pltpu.async_copy(
                    table_hbm.at[idx_v.at[j]],
                    rows_v.at[pl.ds(j * _IDXW, _IDXW)],
                    sem,
                )
            )
        for c in copies:
            c.wait()
        pltpu.sync_copy(rows_v, out_hbm.at[pl.ds(out0 + i * _CHUNK, _CHUNK)])
        return carry

    lax.fori_loop(0, _NCHUNK, chunk_body, 0)


def kernel(y, table):
    idx = y.reshape(_N // _IDXW, _IDXW)
    out = _emb_lookup(table, idx)
    return out.reshape(_B, _L, _D)


# speedup vs baseline: 1.0370x; 1.0370x over previous
import functools

import jax
import jax.numpy as jnp
from jax import lax
from jax.experimental import pallas as pl
from jax.experimental.pallas import tpu as pltpu
from jax.experimental.pallas import tpu_sc as plsc

_V = 1000000
_D = 64
_B, _L = 4096, 200
_N = _B * _L

_NC = 2
_NS = 16
_NW = _NC * _NS
_PER_W = _N // _NW
_IDXW = 128
_CROWS = 2
_CHUNK = _CROWS * _IDXW
_NCHUNK = _PER_W // _CHUNK

_mesh = plsc.VectorSubcoreMesh(core_axis_name="c", subcore_axis_name="s")


@functools.partial(
    pl.kernel,
    out_type=jax.ShapeDtypeStruct((_N, _D), jnp.float32),
    mesh=_mesh,
    scratch_types=[
        pltpu.VMEM((_CROWS, _IDXW), jnp.int32),
        pltpu.VMEM((_CHUNK, 2 * _D), jnp.float32),
        pltpu.VMEM((_CHUNK, _D), jnp.float32),
        pltpu.SemaphoreType.DMA,
    ],
)
def _emb_lookup(table, idx_hbm, out_hbm, idx_v, rows_v, rows_c, sem):
    wid = lax.axis_index("s") * _NC + lax.axis_index("c")
    row0 = wid * (_PER_W // _IDXW)
    out0 = wid * _PER_W

    def chunk_body(i, carry):
        r = row0 + i * _CROWS
        pltpu.sync_copy(idx_hbm.at[pl.ds(r, _CROWS)], idx_v)
        copies = []
        for j in range(_CROWS):
            copies.append(
                pltpu.async_copy(
                    table.at[idx_v.at[j]],
                    rows_v.at[pl.ds(j * _IDXW, _IDXW)],
                    sem,
                )
            )
        for c in copies:
            c.wait()
        def row_body(r, c2):
            for k in range(_D // 16):
                rows_c[r, pl.ds(16 * k, 16)] = rows_v[r, pl.ds(16 * k, 16)]
            return c2

        lax.fori_loop(0, _CHUNK, row_body, 0)
        pltpu.sync_copy(rows_c, out_hbm.at[pl.ds(out0 + i * _CHUNK, _CHUNK)])
        return carry

    lax.fori_loop(0, _NCHUNK, chunk_body, 0)


def kernel(y, table):
    tpad = jnp.pad(table, ((0, 0), (0, _D)))
    idx = y.reshape(_N // _IDXW, _IDXW)
    out = _emb_lookup(tpad, idx)
    return out.reshape(_B, _L, _D)
